# baseline (speedup 1.0000x reference)
import functools

import jax
import jax.numpy as jnp
from jax import lax
from jax.experimental import pallas as pl
from jax.experimental.pallas import tpu as pltpu
from jax.experimental.pallas import tpu_sc as plsc

N = 10000
K = 16
CIN = 128
COUT = 32
M = 4
D = 144
NW = 32
PPT = 320
NP = NW * PPT
CH = 16
NCH = PPT // CH
BN = 1024


def _prep_body(x_ref, w_ref, u_ref, v_ref, c_ref, adj_ref,
               tab_ref, eut_ref, winv_ref):
    xb = x_ref[...]
    dn = (((1,), (1,)), ((), ()))
    wx = lax.dot_general(xb, w_ref[...], dn,
                         preferred_element_type=jnp.float32)
    sv = lax.dot_general(xb, v_ref[...], dn,
                         preferred_element_type=jnp.float32)
    evx = jnp.exp(sv - jnp.max(sv, axis=1, keepdims=True))
    su = lax.dot_general(u_ref[...], xb, dn,
                         preferred_element_type=jnp.float32)
    su = su + c_ref[...]
    eut_ref[...] = jnp.exp(su - jnp.max(su, axis=0, keepdims=True))
    adjb = adj_ref[...]
    cnt = jnp.sum((adjb != 0).astype(jnp.float32), axis=1, keepdims=True)
    winv_ref[...] = jnp.where(cnt > 0.0, 1.0 / jnp.where(cnt > 0.0, cnt, 1.0), 0.0)
    sel = (lax.broadcasted_iota(jnp.int32, (M, CIN), 1) // COUT
           == lax.broadcasted_iota(jnp.int32, (M, CIN), 0)).astype(jnp.float32)
    rep = lax.dot_general(evx, sel, (((1,), (0,)), ((), ())),
                          preferred_element_type=jnp.float32)
    tab_ref[:, 0:128] = wx * rep
    tab_ref[:, 128:132] = evx
    tab_ref[:, 132:144] = jnp.zeros((BN, 12), jnp.float32)


def _prep(x2, W, u, v, c2, adj2):
    grid = NP // BN
    return pl.pallas_call(
        _prep_body,
        grid=(grid,),
        in_specs=[
            pl.BlockSpec((BN, CIN), lambda i: (i, 0)),
            pl.BlockSpec((CIN, CIN), lambda i: (0, 0)),
            pl.BlockSpec((M, CIN), lambda i: (0, 0)),
            pl.BlockSpec((M, CIN), lambda i: (0, 0)),
            pl.BlockSpec((M, 1), lambda i: (0, 0)),
            pl.BlockSpec((BN, K), lambda i: (i, 0)),
        ],
        out_specs=[
            pl.BlockSpec((BN, D), lambda i: (i, 0)),
            pl.BlockSpec((M, BN), lambda i: (0, i)),
            pl.BlockSpec((BN, 1), lambda i: (i, 0)),
        ],
        out_shape=[
            jax.ShapeDtypeStruct((NP, D), jnp.float32),
            jax.ShapeDtypeStruct((M, NP), jnp.float32),
            jax.ShapeDtypeStruct((NP, 1), jnp.float32),
        ],
    )(x2, W, u, v, c2, adj2)


def _sc_body(tab_hbm, eum_hbm, winv_hbm, adjf_hbm, bias_hbm, out_hbm,
             adj_v, eut_v, winv_v, bias_v, g0, g1, w_buf, acc_buf,
             st0, st1, semg0, semg1):
    cid = lax.axis_index("c")
    sid = lax.axis_index("s")
    wid = sid * 2 + cid
    base = pl.multiple_of(wid * PPT, 8)

    pltpu.sync_copy(adjf_hbm.at[wid], adj_v)
    for m in range(M):
        pltpu.sync_copy(eum_hbm.at[pl.ds(pl.multiple_of(m * NP + base, 8), PPT)],
                        eut_v.at[m])
    pltpu.sync_copy(winv_hbm.at[pl.ds(base, PPT)], winv_v)
    pltpu.sync_copy(bias_hbm, bias_v)

    b0 = bias_v[pl.ds(0, 16)]
    b1 = bias_v[pl.ds(16, 16)]
    iota = lax.iota(jnp.int32, 16)
    rows = [iota * 16 + k for k in range(16)]
    itr = iota * 16

    def start_gather(c, gbuf, sem):
        pltpu.async_copy(tab_hbm.at[adj_v.at[2 * c]], gbuf.at[pl.ds(0, 128)], sem)
        pltpu.async_copy(tab_hbm.at[adj_v.at[2 * c + 1]], gbuf.at[pl.ds(128, 128)], sem)

    def wait_gather(c, gbuf, sem):
        pltpu.make_async_copy(tab_hbm.at[adj_v.at[2 * c]],
                              gbuf.at[pl.ds(0, 128)], sem).wait()
        pltpu.make_async_copy(tab_hbm.at[adj_v.at[2 * c + 1]],
                              gbuf.at[pl.ds(128, 128)], sem).wait()

    def compute_chunk(c, gbuf, st):
        pbase = pl.multiple_of(c * CH, 8)
        winv_vec = winv_v[pl.ds(pbase, 16)]
        eus = [eut_v[m, pl.ds(pbase, 16)] for m in range(M)]
        zero = jnp.zeros((16,), jnp.float32)
        for o in range(COUT):
            acc_buf[pl.ds(o * 16, 16)] = zero
        for k in range(16):
            ev = [plsc.load_gather(gbuf, [rows[k], jnp.full((16,), 128 + m, jnp.int32)])
                  for m in range(M)]
            E = ev[0] * eus[0] + ev[1] * eus[1] + ev[2] * eus[2] + ev[3] * eus[3]
            r = winv_vec / E
            for m in range(M):
                w_buf[pl.ds((m * 16 + k) * 16, 16)] = eus[m] * r
        for m in range(M):
            wr = [w_buf[pl.ds((m * 16 + k) * 16, 16)] for k in range(16)]

            @pl.loop(0, COUT, unroll=2)
            def _j(j, m=m, wr=wr):
                col = jnp.broadcast_to(jnp.int32(m * COUT) + j, (16,))
                parts = []
                for q in range(4):
                    a = wr[4 * q] * plsc.load_gather(gbuf, [rows[4 * q], col])
                    for k in range(4 * q + 1, 4 * q + 4):
                        a = a + wr[k] * plsc.load_gather(gbuf, [rows[k], col])
                    parts.append(a)
                acc = (parts[0] + parts[1]) + (parts[2] + parts[3])
                plsc.addupdate(acc_buf.at[pl.ds(pl.multiple_of(j * 16, 8), 16)], acc)
        for p in range(16):
            st[p, pl.ds(0, 16)] = plsc.load_gather(acc_buf, [itr + p]) + b0
            st[p, pl.ds(16, 16)] = plsc.load_gather(acc_buf, [itr + (256 + p)]) + b1
        pltpu.sync_copy(st, out_hbm.at[pl.ds(base + pbase, 16)])

    start_gather(0, g0, semg0)

    @pl.loop(0, NCH // 2)
    def _t(t):
        c0 = 2 * t
        c1 = 2 * t + 1
        start_gather(c1, g1, semg1)
        wait_gather(c0, g0, semg0)
        compute_chunk(c0, g0, st0)

        @pl.when(t < NCH // 2 - 1)
        def _():
            start_gather(c0 + 2, g0, semg0)

        wait_gather(c1, g1, semg1)
        compute_chunk(c1, g1, st1)


_sc_call = functools.partial(
    pl.kernel,
    out_type=jax.ShapeDtypeStruct((NP, COUT), jnp.float32),
    mesh=plsc.VectorSubcoreMesh(core_axis_name="c", subcore_axis_name="s",
                                num_cores=2, num_subcores=16),
    compiler_params=pltpu.CompilerParams(use_tc_tiling_on_sc=False, needs_layout_passes=False),
    scratch_types=[
        pltpu.VMEM((2 * NCH, 128), jnp.int32),
        pltpu.VMEM((M, PPT), jnp.float32),
        pltpu.VMEM((PPT,), jnp.float32),
        pltpu.VMEM((COUT,), jnp.float32),
        pltpu.VMEM((2 * 128, D), jnp.float32),
        pltpu.VMEM((2 * 128, D), jnp.float32),
        pltpu.VMEM((M * 16 * 16,), jnp.float32),
        pltpu.VMEM((COUT * 16,), jnp.float32),
        pltpu.VMEM((CH, COUT), jnp.float32),
        pltpu.VMEM((CH, COUT), jnp.float32),
        pltpu.SemaphoreType.DMA,
        pltpu.SemaphoreType.DMA,
    ],
)(_sc_body)


def kernel(x, adj, weight, bias, u, v, c):
    x2 = jnp.pad(x[0], ((0, NP - N), (0, 0)))
    W = weight.reshape(M * COUT, CIN)
    c2 = c.reshape(M, 1)
    adj2 = jnp.pad(adj[0], ((0, NP - N), (0, 0)))

    tab_body, eut, winv = _prep(x2, W, u, v, c2, adj2)

    special = jnp.zeros((1, D), jnp.float32).at[0, 128:132].set(1.0)
    table = jnp.concatenate([special, tab_body], axis=0)
    eum = eut.reshape(-1)
    winv_f = winv.reshape(NP)
    adjf = adj2.reshape(NW, 2 * NCH, 128)

    out = _sc_call(table, eum, winv_f, adjf, bias)
    return out[:N][None]


# speedup vs baseline: 1.2192x; 1.2192x over previous
import functools

import jax
import jax.numpy as jnp
from jax import lax
from jax.experimental import pallas as pl
from jax.experimental.pallas import tpu as pltpu
from jax.experimental.pallas import tpu_sc as plsc

N = 10000
K = 16
CIN = 128
COUT = 32
M = 4
D = 144
NW = 32
PPT = 320
NP = NW * PPT
CH = 16
NCH = PPT // CH
BN = 1024


def _prep_body(x_ref, w_ref, u_ref, v_ref, c_ref, adj_ref,
               tab_ref, eut_ref, winv_ref):
    xb = x_ref[...]
    dn = (((1,), (1,)), ((), ()))
    wx = lax.dot_general(xb, w_ref[...], dn,
                         preferred_element_type=jnp.float32)
    sv = lax.dot_general(xb, v_ref[...], dn,
                         preferred_element_type=jnp.float32)
    evx = jnp.exp(sv - jnp.max(sv, axis=1, keepdims=True))
    su = lax.dot_general(u_ref[...], xb, dn,
                         preferred_element_type=jnp.float32)
    su = su + c_ref[...]
    eut_ref[...] = jnp.exp(su - jnp.max(su, axis=0, keepdims=True))
    adjb = adj_ref[...]
    cnt = jnp.sum((adjb != 0).astype(jnp.float32), axis=1, keepdims=True)
    winv_ref[...] = jnp.where(cnt > 0.0, 1.0 / jnp.where(cnt > 0.0, cnt, 1.0), 0.0)
    sel = (lax.broadcasted_iota(jnp.int32, (M, CIN), 1) // COUT
           == lax.broadcasted_iota(jnp.int32, (M, CIN), 0)).astype(jnp.float32)
    rep = lax.dot_general(evx, sel, (((1,), (0,)), ((), ())),
                          preferred_element_type=jnp.float32)
    tab_ref[:, 0:128] = wx * rep
    tab_ref[:, 128:132] = evx
    tab_ref[:, 132:136] = evx
    tab_ref[:, 136:140] = evx
    tab_ref[:, 140:144] = evx


def _prep(x2, W, u, v, c2, adj2):
    grid = NP // BN
    return pl.pallas_call(
        _prep_body,
        grid=(grid,),
        in_specs=[
            pl.BlockSpec((BN, CIN), lambda i: (i, 0)),
            pl.BlockSpec((CIN, CIN), lambda i: (0, 0)),
            pl.BlockSpec((M, CIN), lambda i: (0, 0)),
            pl.BlockSpec((M, CIN), lambda i: (0, 0)),
            pl.BlockSpec((M, 1), lambda i: (0, 0)),
            pl.BlockSpec((BN, K), lambda i: (i, 0)),
        ],
        out_specs=[
            pl.BlockSpec((BN, D), lambda i: (i, 0)),
            pl.BlockSpec((M, BN), lambda i: (0, i)),
            pl.BlockSpec((BN, 1), lambda i: (i, 0)),
        ],
        out_shape=[
            jax.ShapeDtypeStruct((NP, D), jnp.float32),
            jax.ShapeDtypeStruct((M, NP), jnp.float32),
            jax.ShapeDtypeStruct((NP, 1), jnp.float32),
        ],
    )(x2, W, u, v, c2, adj2)


def _sc_body(tab_hbm, eum_hbm, winv_hbm, adjf_hbm, bias_hbm, out_hbm,
             adj_v, eut_v, winv_v, bias_v, g0, g1, w_buf, acc_buf,
             st0, st1, semg0, semg1):
    cid = lax.axis_index("c")
    sid = lax.axis_index("s")
    wid = sid * 2 + cid
    base = pl.multiple_of(wid * PPT, 8)

    pltpu.sync_copy(adjf_hbm.at[wid], adj_v)
    for m in range(M):
        pltpu.sync_copy(eum_hbm.at[pl.ds(pl.multiple_of(m * NP + base, 8), PPT)],
                        eut_v.at[m])
    pltpu.sync_copy(winv_hbm.at[pl.ds(base, PPT)], winv_v)
    pltpu.sync_copy(bias_hbm, bias_v)

    b0 = bias_v[pl.ds(0, 16)]
    b1 = bias_v[pl.ds(16, 16)]
    iota = lax.iota(jnp.int32, 16)
    rows = [iota * 16 + k for k in range(16)]
    pm4 = iota & 3
    ecol = [128 + ((iota + r) & 15) for r in range(4)]

    def start_gather(c, gbuf, sem):
        pltpu.async_copy(tab_hbm.at[adj_v.at[2 * c]], gbuf.at[pl.ds(0, 128)], sem)
        pltpu.async_copy(tab_hbm.at[adj_v.at[2 * c + 1]], gbuf.at[pl.ds(128, 128)], sem)

    def wait_gather(c, gbuf, sem):
        pltpu.make_async_copy(tab_hbm.at[adj_v.at[2 * c]],
                              gbuf.at[pl.ds(0, 128)], sem).wait()
        pltpu.make_async_copy(tab_hbm.at[adj_v.at[2 * c + 1]],
                              gbuf.at[pl.ds(128, 128)], sem).wait()

    def compute_chunk(c, gbuf, st):
        pbase = pl.multiple_of(c * CH, 8)
        winv_vec = winv_v[pl.ds(pbase, 16)]
        eus = [eut_v[m, pl.ds(pbase, 16)] for m in range(M)]
        zero = jnp.zeros((16,), jnp.float32)
        for o in range(COUT):
            acc_buf[pl.ds(o * 16, 16)] = zero
        eur = []
        for r in range(4):
            sm = (pm4 + r) & 3
            eur.append(jnp.where(sm == 0, eus[0],
                       jnp.where(sm == 1, eus[1],
                       jnp.where(sm == 2, eus[2], eus[3]))))
        for k in range(16):
            ev = [plsc.load_gather(gbuf, [rows[k], ecol[r]]) for r in range(4)]
            E = (ev[0] * eur[0] + ev[1] * eur[1]) + (ev[2] * eur[2] + ev[3] * eur[3])
            r = winv_vec / E
            for m in range(M):
                w_buf[pl.ds((m * 16 + k) * 16, 16)] = eus[m] * r
        for m in range(M):
            wr = [w_buf[pl.ds((m * 16 + k) * 16, 16)] for k in range(16)]

            @pl.loop(0, COUT, unroll=2)
            def _j(j, m=m, wr=wr):
                t = (j + iota) & 31
                cvec = t + m * COUT
                sidx = (t << 4) + iota
                parts = []
                for q in range(4):
                    a = wr[4 * q] * plsc.load_gather(gbuf, [rows[4 * q], cvec])
                    for k in range(4 * q + 1, 4 * q + 4):
                        a = a + wr[k] * plsc.load_gather(gbuf, [rows[k], cvec])
                    parts.append(a)
                acc = (parts[0] + parts[1]) + (parts[2] + parts[3])
                plsc.addupdate_scatter(acc_buf, [sidx], acc)
        for p in range(16):
            t16 = (p + iota) & 15
            v0 = plsc.load_gather(acc_buf, [(iota << 4) + t16]) + b0
            v1 = plsc.load_gather(acc_buf, [((iota + 16) << 4) + t16]) + b1
            plsc.store_scatter(st, [t16, iota], v0)
            plsc.store_scatter(st, [t16, iota + 16], v1)
        pltpu.sync_copy(st, out_hbm.at[pl.ds(base + pbase, 16)])

    start_gather(0, g0, semg0)

    @pl.loop(0, NCH // 2)
    def _t(t):
        c0 = 2 * t
        c1 = 2 * t + 1
        start_gather(c1, g1, semg1)
        wait_gather(c0, g0, semg0)
        compute_chunk(c0, g0, st0)

        @pl.when(t < NCH // 2 - 1)
        def _():
            start_gather(c0 + 2, g0, semg0)

        wait_gather(c1, g1, semg1)
        compute_chunk(c1, g1, st1)


_sc_call = functools.partial(
    pl.kernel,
    out_type=jax.ShapeDtypeStruct((NP, COUT), jnp.float32),
    mesh=plsc.VectorSubcoreMesh(core_axis_name="c", subcore_axis_name="s",
                                num_cores=2, num_subcores=16),
    compiler_params=pltpu.CompilerParams(use_tc_tiling_on_sc=False, needs_layout_passes=False),
    scratch_types=[
        pltpu.VMEM((2 * NCH, 128), jnp.int32),
        pltpu.VMEM((M, PPT), jnp.float32),
        pltpu.VMEM((PPT,), jnp.float32),
        pltpu.VMEM((COUT,), jnp.float32),
        pltpu.VMEM((2 * 128, D), jnp.float32),
        pltpu.VMEM((2 * 128, D), jnp.float32),
        pltpu.VMEM((M * 16 * 16,), jnp.float32),
        pltpu.VMEM((COUT * 16,), jnp.float32),
        pltpu.VMEM((CH, COUT), jnp.float32),
        pltpu.VMEM((CH, COUT), jnp.float32),
        pltpu.SemaphoreType.DMA,
        pltpu.SemaphoreType.DMA,
    ],
)(_sc_body)


def kernel(x, adj, weight, bias, u, v, c):
    x2 = jnp.pad(x[0], ((0, NP - N), (0, 0)))
    W = weight.reshape(M * COUT, CIN)
    c2 = c.reshape(M, 1)
    adj2 = jnp.pad(adj[0], ((0, NP - N), (0, 0)))

    tab_body, eut, winv = _prep(x2, W, u, v, c2, adj2)

    special = jnp.zeros((1, D), jnp.float32).at[0, 128:144].set(1.0)
    table = jnp.concatenate([special, tab_body], axis=0)
    eum = eut.reshape(-1)
    winv_f = winv.reshape(NP)
    adjf = adj2.reshape(NW, 2 * NCH, 128)

    out = _sc_call(table, eum, winv_f, adjf, bias)
    return out[:N][None]
